# baseline (speedup 1.0000x reference)
import functools

import jax
import jax.numpy as jnp
from jax import lax
from jax.experimental import pallas as pl
from jax.experimental.pallas import tpu as pltpu
from jax.experimental.pallas import tpu_sc as plsc

_N = 10000
_E = 320000
_D = 128
_H = 64
_G = 64
_NC = 2
_NS = 16
_NW = _NC * _NS
_NPAD = 10240
_JUNK = 10200
_CH = 128
_NCHUNK = 80
_EPT = _CH * _NCHUNK
_SLOTS = _NW * _EPT

_mesh = plsc.VectorSubcoreMesh(core_axis_name="c", subcore_axis_name="s")


@functools.partial(
    pl.kernel,
    out_type=jax.ShapeDtypeStruct((_NC, _NPAD, 16), jnp.float32),
    mesh=_mesh,
    compiler_params=pltpu.CompilerParams(needs_layout_passes=False,
                                         use_tc_tiling_on_sc=False),
    scratch_types=[
        pltpu.VMEM_SHARED((_NPAD, 16), jnp.float32),
        pltpu.VMEM((_NCHUNK, _CH), jnp.int32),
        pltpu.VMEM((_CH, 16), jnp.float32),
        pltpu.VMEM((_CH, 16), jnp.float32),
    ],
)
def _sc_hist(dst_hbm, hist_hbm, acc_sh, idx_v, ones_v, zbuf):
    c = lax.axis_index("c")
    s = lax.axis_index("s")
    wid = s * _NC + c

    def _fill(i, carry):
        ones_v[i, pl.ds(0, 16)] = jnp.ones((16,), jnp.float32)
        zbuf[i, pl.ds(0, 16)] = jnp.zeros((16,), jnp.float32)
        return carry

    lax.fori_loop(0, _CH, _fill, 0)
    for k in range(_NPAD // _NS // _CH):
        pltpu.sync_copy(zbuf, acc_sh.at[pl.ds(s * (_NPAD // _NS) + k * _CH, _CH)])
    plsc.subcore_barrier()
    pltpu.sync_copy(dst_hbm.at[wid], idx_v)

    def _chunk(j, carry):
        pltpu.sync_copy(ones_v, acc_sh.at[idx_v.at[j]], add=True)
        return carry

    lax.fori_loop(0, _NCHUNK, _chunk, 0)
    plsc.subcore_barrier()
    for k in range(_NPAD // _NS // _CH):
        r0 = s * (_NPAD // _NS) + k * _CH
        pltpu.sync_copy(acc_sh.at[pl.ds(r0, _CH)],
                        hist_hbm.at[c].at[pl.ds(r0, _CH)])


@functools.partial(
    pl.kernel,
    out_type=jax.ShapeDtypeStruct((_NC, _NPAD, _H), jnp.float32),
    mesh=_mesh,
    compiler_params=pltpu.CompilerParams(needs_layout_passes=False,
                                         use_tc_tiling_on_sc=False),
    scratch_types=[
        pltpu.VMEM_SHARED((_NPAD, _H), jnp.float32),
        pltpu.VMEM((_NCHUNK, _CH), jnp.int32),
        pltpu.VMEM((_NCHUNK, _CH), jnp.int32),
        pltpu.VMEM((_CH, _H), jnp.float32),
        pltpu.VMEM((_CH, _H), jnp.float32),
        pltpu.VMEM((_CH, _H), jnp.float32),
        pltpu.SemaphoreType.DMA,
        pltpu.SemaphoreType.DMA,
        pltpu.SemaphoreType.DMA,
        pltpu.SemaphoreType.DMA,
    ],
)
def _sc_agg(z_hbm, src_hbm, dst_hbm, part_hbm, acc_sh, src_v, dst_v, r0, r1,
            zbuf, sg0, sg1, ss0, ss1):
    c = lax.axis_index("c")
    s = lax.axis_index("s")
    wid = s * _NC + c

    def _zero(i, carry):
        zbuf[i // (_H // 16), pl.ds((i % (_H // 16)) * 16, 16)] = jnp.zeros(
            (16,), jnp.float32)
        return carry

    lax.fori_loop(0, _CH * _H // 16, _zero, 0)
    for k in range(_NPAD // _NS // _CH):
        pltpu.sync_copy(zbuf, acc_sh.at[pl.ds(s * (_NPAD // _NS) + k * _CH, _CH)])
    plsc.subcore_barrier()

    pltpu.sync_copy(src_hbm.at[wid], src_v)
    pltpu.sync_copy(dst_hbm.at[wid], dst_v)

    def _chunk(j, carry):
        pltpu.async_copy(z_hbm.at[src_v.at[j]], r0, sg0).wait()
        pltpu.sync_copy(r0, acc_sh.at[dst_v.at[j]], add=True)
        return carry

    lax.fori_loop(0, _NCHUNK, _chunk, 0)
    plsc.subcore_barrier()
    for k in range(_NPAD // _NS // _CH):
        r0 = s * (_NPAD // _NS) + k * _CH
        pltpu.sync_copy(acc_sh.at[pl.ds(r0, _CH)],
                        part_hbm.at[c].at[pl.ds(r0, _CH)])


def _tc_dis_body(hist_ref, dis_ref):
    p = hist_ref[...]
    deg = p[0, :, 0] + p[1, :, 0] + 1.0
    dis_ref[...] = 1.0 / jnp.sqrt(deg)


_tc_dis = pl.pallas_call(
    _tc_dis_body, out_shape=jax.ShapeDtypeStruct((_NPAD,), jnp.float32))


def _mm_default(a, b):
    return jnp.dot(a, b, precision=lax.Precision.HIGHEST,
                   preferred_element_type=jnp.float32)


def _tc_z1_body(x_ref, w_ref, dis_ref, z_ref):
    xw = _mm_default(x_ref[...], w_ref[...])
    z_ref[...] = xw * dis_ref[...]


_tc_z1 = pl.pallas_call(
    _tc_z1_body, out_shape=jax.ShapeDtypeStruct((_N, _H), jnp.float32))


def _tc_prestats_body(p_ref, z_ref, dis_ref, b_ref, batchr_ref, h_ref,
                      mean_ref, cnt_ref):
    p = p_ref[...]
    agg = p[0, :_N, :] + p[1, :_N, :] + z_ref[...]
    h = dis_ref[...] * agg + b_ref[...]
    h_ref[...] = h
    iota_g = lax.broadcasted_iota(jnp.int32, (_G, _N), 0)
    oht = jnp.where(batchr_ref[...] == iota_g, 1.0, 0.0).astype(jnp.float32)
    cnt = jnp.maximum(_mm_default(oht, jnp.ones((_N, 1), jnp.float32)), 1.0)
    mean_ref[...] = _mm_default(oht, h) / cnt
    cnt_ref[...] = cnt


_tc_prestats = pl.pallas_call(
    _tc_prestats_body,
    out_shape=(jax.ShapeDtypeStruct((_N, _H), jnp.float32),
               jax.ShapeDtypeStruct((_G, _H), jnp.float32),
               jax.ShapeDtypeStruct((_G, 1), jnp.float32)))


def _tc_centervar_body(h_ref, mean_ref, cnt_ref, ms_ref, batch_ref,
                       batchr_ref, out_ref, std_ref):
    iota_n = lax.broadcasted_iota(jnp.int32, (_N, _G), 1)
    onehot = jnp.where(batch_ref[...] == iota_n, 1.0, 0.0).astype(jnp.float32)
    o = h_ref[...] - ms_ref[...] * _mm_default(onehot, mean_ref[...])
    out_ref[...] = o
    iota_g = lax.broadcasted_iota(jnp.int32, (_G, _N), 0)
    oht = jnp.where(batchr_ref[...] == iota_g, 1.0, 0.0).astype(jnp.float32)
    var = _mm_default(oht, o * o) / cnt_ref[...]
    std_ref[...] = jnp.sqrt(var + 1e-5)


_tc_centervar = pl.pallas_call(
    _tc_centervar_body,
    out_shape=(jax.ShapeDtypeStruct((_N, _H), jnp.float32),
               jax.ShapeDtypeStruct((_G, _H), jnp.float32)))


def _tc_apply_body(out_ref, std_ref, w_ref, bias_ref, batch_ref, w2_ref,
                   dis_ref, z2_ref):
    iota_n = lax.broadcasted_iota(jnp.int32, (_N, _G), 1)
    onehot = jnp.where(batch_ref[...] == iota_n, 1.0, 0.0).astype(jnp.float32)
    stdb = _mm_default(onehot, std_ref[...])
    res = w_ref[...] * (out_ref[...] / stdb) + bias_ref[...]
    res = jnp.maximum(res, 0.0)
    z2_ref[...] = _mm_default(res, w2_ref[...]) * dis_ref[...]


_tc_apply = pl.pallas_call(
    _tc_apply_body, out_shape=jax.ShapeDtypeStruct((_N, _H), jnp.float32))


def _tc_applypool_body(out_ref, std_ref, w_ref, bias_ref, batch_ref,
                       batchr_ref, cnt_ref, g_ref):
    iota_n = lax.broadcasted_iota(jnp.int32, (_N, _G), 1)
    onehot = jnp.where(batch_ref[...] == iota_n, 1.0, 0.0).astype(jnp.float32)
    stdb = _mm_default(onehot, std_ref[...])
    res = w_ref[...] * (out_ref[...] / stdb) + bias_ref[...]
    res = jnp.maximum(res, 0.0)
    iota_g = lax.broadcasted_iota(jnp.int32, (_G, _N), 0)
    oht = jnp.where(batchr_ref[...] == iota_g, 1.0, 0.0).astype(jnp.float32)
    g_ref[...] = _mm_default(oht, res) / cnt_ref[...]


_tc_applypool = pl.pallas_call(
    _tc_applypool_body, out_shape=jax.ShapeDtypeStruct((_G, _H), jnp.float32))


def _graph_norm_tc(p, z, dis2d, b_row, w_row, bias_row, ms_row, batch2d,
                   batch_row, w2=None):
    h, mean, cnt = _tc_prestats(p, z, dis2d, b_row, batch_row)
    out, std = _tc_centervar(h, mean, cnt, ms_row, batch2d, batch_row)
    if w2 is None:
        return _tc_applypool(out, std, w_row, bias_row, batch2d, batch_row,
                             cnt)
    return _tc_apply(out, std, w_row, bias_row, batch2d, w2, dis2d)


def kernel(x, edge_index, batch, W1, b1, gn1_w, gn1_b, gn1_ms,
           W2, b2, gn2_w, gn2_b, gn2_ms):
    src = edge_index[0].astype(jnp.int32)
    dst = edge_index[1].astype(jnp.int32)
    pad = _SLOTS - _E
    src3 = jnp.concatenate([src, jnp.zeros((pad,), jnp.int32)]).reshape(
        _NW, _NCHUNK, _CH)
    junk = _N + jnp.arange(pad, dtype=jnp.int32) % (_NPAD - _N)
    dst3 = jnp.concatenate([dst, junk]).reshape(_NW, _NCHUNK, _CH)
    batch2d = batch.astype(jnp.int32).reshape(_N, 1)
    batch_row = batch.astype(jnp.int32).reshape(1, _N)

    hist = _sc_hist(dst3)
    dis2d = _tc_dis(hist)[:_N].reshape(_N, 1)
    z1 = _tc_z1(x, W1, dis2d)
    p1 = _sc_agg(z1, src3, dst3)
    z2 = _graph_norm_tc(p1, z1, dis2d, b1.reshape(1, _H), gn1_w.reshape(1, _H),
                        gn1_b.reshape(1, _H), gn1_ms.reshape(1, _H), batch2d,
                        batch_row, W2)
    p2 = _sc_agg(z2, src3, dst3)
    g = _graph_norm_tc(p2, z2, dis2d, b2.reshape(1, _H), gn2_w.reshape(1, _H),
                       gn2_b.reshape(1, _H), gn2_ms.reshape(1, _H), batch2d,
                       batch_row)
    return g


# speedup vs baseline: 1.2433x; 1.2433x over previous
import functools

import jax
import jax.numpy as jnp
from jax import lax
from jax.experimental import pallas as pl
from jax.experimental.pallas import tpu as pltpu
from jax.experimental.pallas import tpu_sc as plsc

_N = 10000
_E = 320000
_D = 128
_H = 64
_G = 64
_NC = 2
_NS = 16
_NW = _NC * _NS
_NPAD = 10240
_JUNK = 10200
_CH = 128
_NCHUNK = 79
_EPT = _CH * _NCHUNK
_SLOTS = _NW * _EPT

_mesh = plsc.VectorSubcoreMesh(core_axis_name="c", subcore_axis_name="s")


@functools.partial(
    pl.kernel,
    out_type=jax.ShapeDtypeStruct((_NC, _NPAD, 16), jnp.float32),
    mesh=_mesh,
    compiler_params=pltpu.CompilerParams(needs_layout_passes=False,
                                         use_tc_tiling_on_sc=False),
    scratch_types=[
        pltpu.VMEM_SHARED((_NPAD, 16), jnp.float32),
        pltpu.VMEM((_NCHUNK, _CH), jnp.int32),
        pltpu.VMEM((_CH, 16), jnp.float32),
        pltpu.VMEM((_CH, 16), jnp.float32),
    ],
)
def _sc_hist(dst_hbm, hist_hbm, acc_sh, idx_v, ones_v, zbuf):
    c = lax.axis_index("c")
    s = lax.axis_index("s")
    wid = s * _NC + c

    def _fill(i, carry):
        ones_v[i, pl.ds(0, 16)] = jnp.ones((16,), jnp.float32)
        zbuf[i, pl.ds(0, 16)] = jnp.zeros((16,), jnp.float32)
        return carry

    lax.fori_loop(0, _CH, _fill, 0)
    for k in range(_NPAD // _NS // _CH):
        pltpu.sync_copy(zbuf, acc_sh.at[pl.ds(s * (_NPAD // _NS) + k * _CH, _CH)])
    plsc.subcore_barrier()
    pltpu.sync_copy(dst_hbm.at[wid], idx_v)

    def _chunk(j, carry):
        pltpu.sync_copy(ones_v, acc_sh.at[idx_v.at[j]], add=True)
        return carry

    lax.fori_loop(0, _NCHUNK, _chunk, 0)
    plsc.subcore_barrier()
    for k in range(_NPAD // _NS // _CH):
        r0 = s * (_NPAD // _NS) + k * _CH
        pltpu.sync_copy(acc_sh.at[pl.ds(r0, _CH)],
                        hist_hbm.at[c].at[pl.ds(r0, _CH)])


@functools.partial(
    pl.kernel,
    out_type=jax.ShapeDtypeStruct((_NC, _NPAD, _H), jnp.float32),
    mesh=_mesh,
    compiler_params=pltpu.CompilerParams(needs_layout_passes=False,
                                         use_tc_tiling_on_sc=False),
    scratch_types=[
        pltpu.VMEM_SHARED((_NPAD, _H), jnp.float32),
        pltpu.VMEM((_NCHUNK, _CH), jnp.int32),
        pltpu.VMEM((_NCHUNK, _CH), jnp.int32),
        pltpu.VMEM((_CH, _H), jnp.float32),
        pltpu.VMEM((_CH, _H), jnp.float32),
        pltpu.SemaphoreType.DMA,
    ],
)
def _sc_agg(z_hbm, src_hbm, dst_hbm, part_hbm, acc_sh, src_v, dst_v, r0,
            zbuf, sg0):
    c = lax.axis_index("c")
    s = lax.axis_index("s")
    wid = s * _NC + c

    def _zero(i, carry):
        zbuf[i // (_H // 16), pl.ds((i % (_H // 16)) * 16, 16)] = jnp.zeros(
            (16,), jnp.float32)
        return carry

    lax.fori_loop(0, _CH * _H // 16, _zero, 0)
    for k in range(_NPAD // _NS // _CH):
        pltpu.sync_copy(zbuf, acc_sh.at[pl.ds(s * (_NPAD // _NS) + k * _CH, _CH)])
    plsc.subcore_barrier()

    pltpu.sync_copy(src_hbm.at[wid], src_v)
    pltpu.sync_copy(dst_hbm.at[wid], dst_v)

    def _chunk(j, carry):
        pltpu.async_copy(z_hbm.at[src_v.at[j]], r0, sg0).wait()
        pltpu.sync_copy(r0, acc_sh.at[dst_v.at[j]], add=True)
        return carry

    lax.fori_loop(0, _NCHUNK, _chunk, 0)
    plsc.subcore_barrier()
    for k in range(_NPAD // _NS // _CH):
        r0 = s * (_NPAD // _NS) + k * _CH
        pltpu.sync_copy(acc_sh.at[pl.ds(r0, _CH)],
                        part_hbm.at[c].at[pl.ds(r0, _CH)])


def _tc_dis_body(hist_ref, dis_ref):
    p = hist_ref[...]
    deg = p[0, :, 0] + p[1, :, 0] + 1.0
    dis_ref[...] = 1.0 / jnp.sqrt(deg)


_tc_dis = pl.pallas_call(
    _tc_dis_body, out_shape=jax.ShapeDtypeStruct((_NPAD,), jnp.float32))


def _mm_default(a, b):
    return jnp.dot(a, b, precision=lax.Precision.HIGHEST,
                   preferred_element_type=jnp.float32)


def _tc_z1_body(x_ref, w_ref, dis_ref, z_ref):
    xw = _mm_default(x_ref[...], w_ref[...])
    z_ref[...] = xw * dis_ref[...]


_tc_z1 = pl.pallas_call(
    _tc_z1_body, out_shape=jax.ShapeDtypeStruct((_N, _H), jnp.float32))


def _tc_prestats_body(p_ref, z_ref, dis_ref, b_ref, batchr_ref, h_ref,
                      mean_ref, cnt_ref):
    p = p_ref[...]
    agg = p[0, :_N, :] + p[1, :_N, :] + z_ref[...]
    h = dis_ref[...] * agg + b_ref[...]
    h_ref[...] = h
    iota_g = lax.broadcasted_iota(jnp.int32, (_G, _N), 0)
    oht = jnp.where(batchr_ref[...] == iota_g, 1.0, 0.0).astype(jnp.float32)
    cnt = jnp.maximum(_mm_default(oht, jnp.ones((_N, 1), jnp.float32)), 1.0)
    mean_ref[...] = _mm_default(oht, h) / cnt
    cnt_ref[...] = cnt


_tc_prestats = pl.pallas_call(
    _tc_prestats_body,
    out_shape=(jax.ShapeDtypeStruct((_N, _H), jnp.float32),
               jax.ShapeDtypeStruct((_G, _H), jnp.float32),
               jax.ShapeDtypeStruct((_G, 1), jnp.float32)))


def _tc_centervar_body(h_ref, mean_ref, cnt_ref, ms_ref, batch_ref,
                       batchr_ref, out_ref, std_ref):
    iota_n = lax.broadcasted_iota(jnp.int32, (_N, _G), 1)
    onehot = jnp.where(batch_ref[...] == iota_n, 1.0, 0.0).astype(jnp.float32)
    o = h_ref[...] - ms_ref[...] * _mm_default(onehot, mean_ref[...])
    out_ref[...] = o
    iota_g = lax.broadcasted_iota(jnp.int32, (_G, _N), 0)
    oht = jnp.where(batchr_ref[...] == iota_g, 1.0, 0.0).astype(jnp.float32)
    var = _mm_default(oht, o * o) / cnt_ref[...]
    std_ref[...] = jnp.sqrt(var + 1e-5)


_tc_centervar = pl.pallas_call(
    _tc_centervar_body,
    out_shape=(jax.ShapeDtypeStruct((_N, _H), jnp.float32),
               jax.ShapeDtypeStruct((_G, _H), jnp.float32)))


def _tc_apply_body(out_ref, std_ref, w_ref, bias_ref, batch_ref, w2_ref,
                   dis_ref, z2_ref):
    iota_n = lax.broadcasted_iota(jnp.int32, (_N, _G), 1)
    onehot = jnp.where(batch_ref[...] == iota_n, 1.0, 0.0).astype(jnp.float32)
    stdb = _mm_default(onehot, std_ref[...])
    res = w_ref[...] * (out_ref[...] / stdb) + bias_ref[...]
    res = jnp.maximum(res, 0.0)
    z2_ref[...] = _mm_default(res, w2_ref[...]) * dis_ref[...]


_tc_apply = pl.pallas_call(
    _tc_apply_body, out_shape=jax.ShapeDtypeStruct((_N, _H), jnp.float32))


def _tc_applypool_body(out_ref, std_ref, w_ref, bias_ref, batch_ref,
                       batchr_ref, cnt_ref, g_ref):
    iota_n = lax.broadcasted_iota(jnp.int32, (_N, _G), 1)
    onehot = jnp.where(batch_ref[...] == iota_n, 1.0, 0.0).astype(jnp.float32)
    stdb = _mm_default(onehot, std_ref[...])
    res = w_ref[...] * (out_ref[...] / stdb) + bias_ref[...]
    res = jnp.maximum(res, 0.0)
    iota_g = lax.broadcasted_iota(jnp.int32, (_G, _N), 0)
    oht = jnp.where(batchr_ref[...] == iota_g, 1.0, 0.0).astype(jnp.float32)
    g_ref[...] = _mm_default(oht, res) / cnt_ref[...]


_tc_applypool = pl.pallas_call(
    _tc_applypool_body, out_shape=jax.ShapeDtypeStruct((_G, _H), jnp.float32))


def _graph_norm_tc(p, z, dis2d, b_row, w_row, bias_row, ms_row, batch2d,
                   batch_row, w2=None):
    h, mean, cnt = _tc_prestats(p, z, dis2d, b_row, batch_row)
    out, std = _tc_centervar(h, mean, cnt, ms_row, batch2d, batch_row)
    if w2 is None:
        return _tc_applypool(out, std, w_row, bias_row, batch2d, batch_row,
                             cnt)
    return _tc_apply(out, std, w_row, bias_row, batch2d, w2, dis2d)


def kernel(x, edge_index, batch, W1, b1, gn1_w, gn1_b, gn1_ms,
           W2, b2, gn2_w, gn2_b, gn2_ms):
    src = edge_index[0].astype(jnp.int32)
    dst = edge_index[1].astype(jnp.int32)
    pad = _SLOTS - _E
    src3 = jnp.concatenate([src, jnp.zeros((pad,), jnp.int32)]).reshape(
        _NW, _NCHUNK, _CH)
    junk = _N + jnp.arange(pad, dtype=jnp.int32) % (_NPAD - _N)
    dst3 = jnp.concatenate([dst, junk]).reshape(_NW, _NCHUNK, _CH)
    batch2d = batch.astype(jnp.int32).reshape(_N, 1)
    batch_row = batch.astype(jnp.int32).reshape(1, _N)

    hist = _sc_hist(dst3)
    dis2d = _tc_dis(hist)[:_N].reshape(_N, 1)
    z1 = _tc_z1(x, W1, dis2d)
    p1 = _sc_agg(z1, src3, dst3)
    z2 = _graph_norm_tc(p1, z1, dis2d, b1.reshape(1, _H), gn1_w.reshape(1, _H),
                        gn1_b.reshape(1, _H), gn1_ms.reshape(1, _H), batch2d,
                        batch_row, W2)
    p2 = _sc_agg(z2, src3, dst3)
    g = _graph_norm_tc(p2, z2, dis2d, b2.reshape(1, _H), gn2_w.reshape(1, _H),
                       gn2_b.reshape(1, _H), gn2_ms.reshape(1, _H), batch2d,
                       batch_row)
    return g


# speedup vs baseline: 1.3685x; 1.1007x over previous
import functools

import jax
import jax.numpy as jnp
from jax import lax
from jax.experimental import pallas as pl
from jax.experimental.pallas import tpu as pltpu
from jax.experimental.pallas import tpu_sc as plsc

_N = 10000
_E = 320000
_D = 128
_H = 64
_G = 64
_NC = 2
_NS = 16
_NW = _NC * _NS
_NPAD = 10240
_JUNK = 10200
_CH = 128
_NCHUNK = 79
_EPT = _CH * _NCHUNK
_SLOTS = _NW * _EPT

_mesh = plsc.VectorSubcoreMesh(core_axis_name="c", subcore_axis_name="s")


@functools.partial(
    pl.kernel,
    out_type=jax.ShapeDtypeStruct((_NC, _NPAD, 16), jnp.float32),
    mesh=_mesh,
    compiler_params=pltpu.CompilerParams(needs_layout_passes=False,
                                         use_tc_tiling_on_sc=False),
    scratch_types=[
        pltpu.VMEM_SHARED((_NPAD, 16), jnp.float32),
        pltpu.VMEM((_NCHUNK, _CH), jnp.int32),
        pltpu.VMEM((_CH, 16), jnp.float32),
        pltpu.VMEM((_CH, 16), jnp.float32),
    ],
)
def _sc_hist(dst_hbm, hist_hbm, acc_sh, idx_v, ones_v, zbuf):
    c = lax.axis_index("c")
    s = lax.axis_index("s")
    wid = s * _NC + c

    def _fill(i, carry):
        ones_v[i, pl.ds(0, 16)] = jnp.ones((16,), jnp.float32)
        zbuf[i, pl.ds(0, 16)] = jnp.zeros((16,), jnp.float32)
        return carry

    lax.fori_loop(0, _CH, _fill, 0)
    for k in range(_NPAD // _NS // _CH):
        pltpu.sync_copy(zbuf, acc_sh.at[pl.ds(s * (_NPAD // _NS) + k * _CH, _CH)])
    plsc.subcore_barrier()
    pltpu.sync_copy(dst_hbm.at[wid], idx_v)

    def _chunk(j, carry):
        pltpu.sync_copy(ones_v, acc_sh.at[idx_v.at[j]], add=True)
        return carry

    lax.fori_loop(0, _NCHUNK, _chunk, 0)
    plsc.subcore_barrier()
    for k in range(_NPAD // _NS // _CH):
        r0 = s * (_NPAD // _NS) + k * _CH
        pltpu.sync_copy(acc_sh.at[pl.ds(r0, _CH)],
                        hist_hbm.at[c].at[pl.ds(r0, _CH)])


@functools.partial(
    pl.kernel,
    out_type=jax.ShapeDtypeStruct((_NC, _NPAD, _H), jnp.float32),
    mesh=_mesh,
    compiler_params=pltpu.CompilerParams(needs_layout_passes=False,
                                         use_tc_tiling_on_sc=False),
    scratch_types=[
        pltpu.VMEM_SHARED((_NPAD, _H), jnp.float32),
        pltpu.VMEM((_NCHUNK, _CH), jnp.int32),
        pltpu.VMEM((_NCHUNK, _CH), jnp.int32),
        pltpu.VMEM((_CH, _H), jnp.float32),
        pltpu.VMEM((_CH, _H), jnp.float32),
        pltpu.VMEM((_CH, _H), jnp.float32),
        pltpu.SemaphoreType.DMA,
        pltpu.SemaphoreType.DMA,
    ],
)
def _sc_agg(z_hbm, src_hbm, dst_hbm, part_hbm, acc_sh, src_v, dst_v, r0, r1,
            zbuf, sg0, sg1):
    c = lax.axis_index("c")
    s = lax.axis_index("s")
    wid = s * _NC + c

    def _zero(i, carry):
        zbuf[i // (_H // 16), pl.ds((i % (_H // 16)) * 16, 16)] = jnp.zeros(
            (16,), jnp.float32)
        return carry

    lax.fori_loop(0, _CH * _H // 16, _zero, 0)
    for k in range(_NPAD // _NS // _CH):
        pltpu.sync_copy(zbuf, acc_sh.at[pl.ds(s * (_NPAD // _NS) + k * _CH, _CH)])
    plsc.subcore_barrier()

    pltpu.sync_copy(src_hbm.at[wid], src_v)
    pltpu.sync_copy(dst_hbm.at[wid], dst_v)

    npair = _NCHUNK // 2
    pltpu.async_copy(z_hbm.at[src_v.at[0]], r0, sg0)

    def _pair(j, carry):
        pltpu.make_async_copy(z_hbm.at[src_v.at[2 * j]], r0, sg0).wait()
        pltpu.async_copy(z_hbm.at[src_v.at[2 * j + 1]], r1, sg1)
        pltpu.sync_copy(r0, acc_sh.at[dst_v.at[2 * j]], add=True)
        pltpu.make_async_copy(z_hbm.at[src_v.at[2 * j + 1]], r1, sg1).wait()
        pltpu.async_copy(z_hbm.at[src_v.at[2 * j + 2]], r0, sg0)
        pltpu.sync_copy(r1, acc_sh.at[dst_v.at[2 * j + 1]], add=True)
        return carry

    lax.fori_loop(0, npair, _pair, 0)
    pltpu.make_async_copy(z_hbm.at[src_v.at[_NCHUNK - 1]], r0, sg0).wait()
    pltpu.sync_copy(r0, acc_sh.at[dst_v.at[_NCHUNK - 1]], add=True)
    plsc.subcore_barrier()
    for k in range(_NPAD // _NS // _CH):
        r0 = s * (_NPAD // _NS) + k * _CH
        pltpu.sync_copy(acc_sh.at[pl.ds(r0, _CH)],
                        part_hbm.at[c].at[pl.ds(r0, _CH)])


def _tc_dis_body(hist_ref, dis_ref):
    p = hist_ref[...]
    deg = p[0, :, 0] + p[1, :, 0] + 1.0
    dis_ref[...] = 1.0 / jnp.sqrt(deg)


_tc_dis = pl.pallas_call(
    _tc_dis_body, out_shape=jax.ShapeDtypeStruct((_NPAD,), jnp.float32))


def _mm_default(a, b):
    return jnp.dot(a, b, precision=lax.Precision.HIGHEST,
                   preferred_element_type=jnp.float32)


def _tc_z1_body(x_ref, w_ref, dis_ref, z_ref):
    xw = _mm_default(x_ref[...], w_ref[...])
    z_ref[...] = xw * dis_ref[...]


_tc_z1 = pl.pallas_call(
    _tc_z1_body, out_shape=jax.ShapeDtypeStruct((_N, _H), jnp.float32))


def _tc_prestats_body(p_ref, z_ref, dis_ref, b_ref, batchr_ref, h_ref,
                      mean_ref, cnt_ref):
    p = p_ref[...]
    agg = p[0, :_N, :] + p[1, :_N, :] + z_ref[...]
    h = dis_ref[...] * agg + b_ref[...]
    h_ref[...] = h
    iota_g = lax.broadcasted_iota(jnp.int32, (_G, _N), 0)
    oht = jnp.where(batchr_ref[...] == iota_g, 1.0, 0.0).astype(jnp.float32)
    cnt = jnp.maximum(_mm_default(oht, jnp.ones((_N, 1), jnp.float32)), 1.0)
    mean_ref[...] = _mm_default(oht, h) / cnt
    cnt_ref[...] = cnt


_tc_prestats = pl.pallas_call(
    _tc_prestats_body,
    out_shape=(jax.ShapeDtypeStruct((_N, _H), jnp.float32),
               jax.ShapeDtypeStruct((_G, _H), jnp.float32),
               jax.ShapeDtypeStruct((_G, 1), jnp.float32)))


def _tc_centervar_body(h_ref, mean_ref, cnt_ref, ms_ref, batch_ref,
                       batchr_ref, out_ref, std_ref):
    iota_n = lax.broadcasted_iota(jnp.int32, (_N, _G), 1)
    onehot = jnp.where(batch_ref[...] == iota_n, 1.0, 0.0).astype(jnp.float32)
    o = h_ref[...] - ms_ref[...] * _mm_default(onehot, mean_ref[...])
    out_ref[...] = o
    iota_g = lax.broadcasted_iota(jnp.int32, (_G, _N), 0)
    oht = jnp.where(batchr_ref[...] == iota_g, 1.0, 0.0).astype(jnp.float32)
    var = _mm_default(oht, o * o) / cnt_ref[...]
    std_ref[...] = jnp.sqrt(var + 1e-5)


_tc_centervar = pl.pallas_call(
    _tc_centervar_body,
    out_shape=(jax.ShapeDtypeStruct((_N, _H), jnp.float32),
               jax.ShapeDtypeStruct((_G, _H), jnp.float32)))


def _tc_apply_body(out_ref, std_ref, w_ref, bias_ref, batch_ref, w2_ref,
                   dis_ref, z2_ref):
    iota_n = lax.broadcasted_iota(jnp.int32, (_N, _G), 1)
    onehot = jnp.where(batch_ref[...] == iota_n, 1.0, 0.0).astype(jnp.float32)
    stdb = _mm_default(onehot, std_ref[...])
    res = w_ref[...] * (out_ref[...] / stdb) + bias_ref[...]
    res = jnp.maximum(res, 0.0)
    z2_ref[...] = _mm_default(res, w2_ref[...]) * dis_ref[...]


_tc_apply = pl.pallas_call(
    _tc_apply_body, out_shape=jax.ShapeDtypeStruct((_N, _H), jnp.float32))


def _tc_applypool_body(out_ref, std_ref, w_ref, bias_ref, batch_ref,
                       batchr_ref, cnt_ref, g_ref):
    iota_n = lax.broadcasted_iota(jnp.int32, (_N, _G), 1)
    onehot = jnp.where(batch_ref[...] == iota_n, 1.0, 0.0).astype(jnp.float32)
    stdb = _mm_default(onehot, std_ref[...])
    res = w_ref[...] * (out_ref[...] / stdb) + bias_ref[...]
    res = jnp.maximum(res, 0.0)
    iota_g = lax.broadcasted_iota(jnp.int32, (_G, _N), 0)
    oht = jnp.where(batchr_ref[...] == iota_g, 1.0, 0.0).astype(jnp.float32)
    g_ref[...] = _mm_default(oht, res) / cnt_ref[...]


_tc_applypool = pl.pallas_call(
    _tc_applypool_body, out_shape=jax.ShapeDtypeStruct((_G, _H), jnp.float32))


def _graph_norm_tc(p, z, dis2d, b_row, w_row, bias_row, ms_row, batch2d,
                   batch_row, w2=None):
    h, mean, cnt = _tc_prestats(p, z, dis2d, b_row, batch_row)
    out, std = _tc_centervar(h, mean, cnt, ms_row, batch2d, batch_row)
    if w2 is None:
        return _tc_applypool(out, std, w_row, bias_row, batch2d, batch_row,
                             cnt)
    return _tc_apply(out, std, w_row, bias_row, batch2d, w2, dis2d)


def kernel(x, edge_index, batch, W1, b1, gn1_w, gn1_b, gn1_ms,
           W2, b2, gn2_w, gn2_b, gn2_ms):
    src = edge_index[0].astype(jnp.int32)
    dst = edge_index[1].astype(jnp.int32)
    pad = _SLOTS - _E
    src3 = jnp.concatenate([src, jnp.zeros((pad,), jnp.int32)]).reshape(
        _NW, _NCHUNK, _CH)
    junk = _N + jnp.arange(pad, dtype=jnp.int32) % (_NPAD - _N)
    dst3 = jnp.concatenate([dst, junk]).reshape(_NW, _NCHUNK, _CH)
    batch2d = batch.astype(jnp.int32).reshape(_N, 1)
    batch_row = batch.astype(jnp.int32).reshape(1, _N)

    hist = _sc_hist(dst3)
    dis2d = _tc_dis(hist)[:_N].reshape(_N, 1)
    z1 = _tc_z1(x, W1, dis2d)
    p1 = _sc_agg(z1, src3, dst3)
    z2 = _graph_norm_tc(p1, z1, dis2d, b1.reshape(1, _H), gn1_w.reshape(1, _H),
                        gn1_b.reshape(1, _H), gn1_ms.reshape(1, _H), batch2d,
                        batch_row, W2)
    p2 = _sc_agg(z2, src3, dst3)
    g = _graph_norm_tc(p2, z2, dis2d, b2.reshape(1, _H), gn2_w.reshape(1, _H),
                       gn2_b.reshape(1, _H), gn2_ms.reshape(1, _H), batch2d,
                       batch_row)
    return g
